# baseline (speedup 1.0000x reference)
import functools

import jax
import jax.numpy as jnp
from jax import lax
from jax.experimental import pallas as pl
from jax.experimental.pallas import tpu as pltpu
from jax.experimental.pallas import tpu_sc as plsc

_NC = 2
_NS = 16
_NW = _NC * _NS
_C = 128
_RING = 2
_L = 128


@functools.lru_cache(maxsize=None)
def _build_transpose(V, D):
    full = V // _L
    rem = V % _L
    base = full // _NW
    pairs = base // 2
    odd = base % 2
    extra = full % _NW
    scale = jnp.float32(float(D) ** -0.5)
    mesh = plsc.VectorSubcoreMesh(core_axis_name="c", subcore_axis_name="s")

    @functools.partial(
        pl.kernel,
        mesh=mesh,
        compiler_params=pltpu.CompilerParams(
            use_tc_tiling_on_sc=True, needs_layout_passes=False),
        out_type=jax.ShapeDtypeStruct((V, _L), jnp.float32),
        scratch_types=(
            [pltpu.VMEM((D, _L), jnp.float32) for _ in range(2)]
            + [pltpu.VMEM((_L, _L), jnp.float32) for _ in range(2)]
            + ([pltpu.VMEM((rem, D), jnp.float32)] if rem else [])
            + [pltpu.SemaphoreType.DMA for _ in range(4)]
        ),
    )
    def body(tab_hbm, *args):
        if rem:
            tail_hbm, w_hbm, in0, in1, wb0, wb1, tbuf, gi0, gi1, so0, so1 = args
        else:
            w_hbm, in0, in1, wb0, wb1, gi0, gi1, so0, so1 = args
        ins, wbs = (in0, in1), (wb0, wb1)
        gsems, ssems = (gi0, gi1), (so0, so1)
        wid = lax.axis_index("s") * _NC + lax.axis_index("c")

        def in_copy(blk, slot):
            off = pl.multiple_of(blk * _L, _L)
            return pltpu.make_async_copy(
                tab_hbm.at[:, pl.ds(off, _L)], ins[slot], gsems[slot])

        def out_copy(row0, n, slot):
            return pltpu.make_async_copy(
                wbs[slot].at[pl.ds(0, n)],
                w_hbm.at[pl.ds(row0, n)], ssems[slot])

        bases = [lax.iota(jnp.int32, 16) + j * 16 for j in range(D // 16)]

        def transpose(in_ref, w_ref):
            for i in range(_L):
                col16 = jnp.full((16,), i, jnp.int32)
                for j in range(D // 16):
                    v = plsc.load_gather(in_ref, [bases[j], col16])
                    w_ref[i, pl.ds(j * 16, 16)] = v * scale

        in_copy(wid, 0).start()
        in_copy(wid + _NW, 1).start()

        def pair_step(g, carry):
            for slot in range(2):
                blk = wid + _NW * (2 * g + slot)
                in_copy(blk, slot).wait()

                @pl.when(g > 0)
                def _():
                    out_copy(0, _L, slot).wait()

                transpose(ins[slot], wbs[slot])
                out_copy(blk * _L, _L, slot).start()

                @pl.when(g + 1 < pairs)
                def _():
                    in_copy(blk + 2 * _NW, slot).start()

            return carry

        lax.fori_loop(0, pairs, pair_step, 0)
        out_copy(0, _L, 0).wait()
        out_copy(0, _L, 1).wait()

        if odd:
            blk = wid + _NW * (2 * pairs)
            in_copy(blk, 0).start()
            in_copy(blk, 0).wait()
            transpose(ins[0], wbs[0])
            out_copy(blk * _L, _L, 0).start()
            out_copy(0, _L, 0).wait()

        if extra:
            @pl.when(wid < extra)
            def _():
                blk = _NW * base + wid
                in_copy(blk, 0).start()
                in_copy(blk, 0).wait()
                transpose(ins[0], wbs[0])
                out_copy(blk * _L, _L, 0).start()
                out_copy(0, _L, 0).wait()

        if rem:
            @pl.when(wid == extra % _NW)
            def _():
                tc = pltpu.make_async_copy(tail_hbm, tbuf, gsems[1])
                tc.start()
                tc.wait()

                @plsc.parallel_loop(0, rem)
                def _(i):
                    for j in range(D // 16):
                        wbs[1][i, pl.ds(j * 16, 16)] = tbuf[i, pl.ds(j * 16, 16)]

                out_copy(V - rem, rem, 1).start()
                out_copy(0, rem, 1).wait()

    return body


@functools.lru_cache(maxsize=None)
def _build_gather(N, D, V):
    per_w = N // _NW
    n_chunks = per_w // _C
    n_super = n_chunks // _RING
    mesh = plsc.VectorSubcoreMesh(core_axis_name="c", subcore_axis_name="s")

    @functools.partial(
        pl.kernel,
        mesh=mesh,
        compiler_params=pltpu.CompilerParams(use_tc_tiling_on_sc=True),
        out_type=jax.ShapeDtypeStruct((N, D), jnp.float32),
        scratch_types=(
            [pltpu.VMEM((per_w,), jnp.int32)]
            + [pltpu.VMEM((_C, _L), jnp.float32) for _ in range(_RING)]
            + [pltpu.VMEM((_C, D), jnp.float32) for _ in range(_RING)]
            + [pltpu.SemaphoreType.DMA for _ in range(2 * _RING)]
        ),
    )
    def body(x_hbm, w_hbm, out_hbm, idx_v, *rest):
        rows = rest[:_RING]
        stage = rest[_RING:2 * _RING]
        gsems = rest[2 * _RING:3 * _RING]
        ssems = rest[3 * _RING:]
        wid = lax.axis_index("s") * _NC + lax.axis_index("c")
        base = wid * per_w

        pltpu.sync_copy(x_hbm.at[pl.ds(base, per_w)], idx_v)

        def g_copy(chunk, slot):
            return pltpu.make_async_copy(
                w_hbm.at[idx_v.at[pl.ds(chunk * _C, _C)]],
                rows[slot], gsems[slot])

        def s_copy(chunk, slot):
            return pltpu.make_async_copy(
                stage[slot],
                out_hbm.at[pl.ds(base + chunk * _C, _C)], ssems[slot])

        def compact(slot):
            rbuf, sbuf = rows[slot], stage[slot]

            @plsc.parallel_loop(0, _C, unroll=8)
            def _(i):
                for j in range(D // 16):
                    sbuf[i, pl.ds(j * 16, 16)] = rbuf[i, pl.ds(j * 16, 16)]

        for slot in range(_RING):
            g_copy(slot, slot).start()

        def super_step(g, carry):
            c0 = g * _RING
            for slot in range(_RING):
                g_copy(c0 + slot, slot).wait()

                @pl.when(g > 0)
                def _():
                    s_copy(c0 + slot, slot).wait()

                compact(slot)
                s_copy(c0 + slot, slot).start()

                @pl.when(g + 1 < n_super)
                def _():
                    g_copy(c0 + _RING + slot, slot).start()

            return carry

        lax.fori_loop(0, n_super, super_step, 0)

        for slot in range(_RING):
            s_copy(n_chunks - _RING + slot, slot).wait()

    return body


def kernel(x, table):
    B, S = x.shape
    V, D = table.shape
    N = B * S
    xf = x.reshape(N).astype(jnp.int32)
    rem = V % _L
    args = (table.T,)
    if rem:
        args += (table[V - rem:, :] * jnp.float32(float(D) ** -0.5),)
    w = _build_transpose(V, D)(*args)
    out = _build_gather(N, D, V)(xf, w)
    return out.reshape(B, S, D)


# speedup vs baseline: 2.8583x; 2.8583x over previous
import functools

import jax
import jax.numpy as jnp
from jax import lax
from jax.experimental import pallas as pl
from jax.experimental.pallas import tpu as pltpu
from jax.experimental.pallas import tpu_sc as plsc

_NC = 2
_NS = 16
_NW = _NC * _NS
_C = 128
_RING = 2
_L = 128


@functools.lru_cache(maxsize=None)
def _build_transpose(V, D):
    full = V // _L
    rem = V % _L
    base = full // _NW
    pairs = base // 2
    odd = base % 2
    extra = full % _NW
    scale = jnp.float32(float(D) ** -0.5)
    mesh = plsc.VectorSubcoreMesh(core_axis_name="c", subcore_axis_name="s")

    @functools.partial(
        pl.kernel,
        mesh=mesh,
        compiler_params=pltpu.CompilerParams(
            use_tc_tiling_on_sc=True, needs_layout_passes=False),
        out_type=jax.ShapeDtypeStruct((V, _L), jnp.float32),
        scratch_types=(
            [pltpu.VMEM((D, _L), jnp.float32) for _ in range(2)]
            + [pltpu.VMEM((_L, _L), jnp.float32) for _ in range(2)]
            + ([pltpu.VMEM((rem, D), jnp.float32)] if rem else [])
            + [pltpu.SemaphoreType.DMA for _ in range(4)]
        ),
    )
    def body(tab_hbm, *args):
        if rem:
            tail_hbm, w_hbm, in0, in1, wb0, wb1, tbuf, gi0, gi1, so0, so1 = args
        else:
            w_hbm, in0, in1, wb0, wb1, gi0, gi1, so0, so1 = args
        ins, wbs = (in0, in1), (wb0, wb1)
        gsems, ssems = (gi0, gi1), (so0, so1)
        wid = lax.axis_index("s") * _NC + lax.axis_index("c")

        def in_copy(blk, slot):
            off = pl.multiple_of(blk * _L, _L)
            return pltpu.make_async_copy(
                tab_hbm.at[:, pl.ds(off, _L)], ins[slot], gsems[slot])

        def out_copy(row0, n, slot):
            return pltpu.make_async_copy(
                wbs[slot].at[pl.ds(0, n)],
                w_hbm.at[pl.ds(row0, n)], ssems[slot])

        bases = [lax.iota(jnp.int32, 16) + j * 16 for j in range(D // 16)]

        def transpose(in_ref, w_ref):
            @plsc.parallel_loop(0, _L, unroll=4)
            def _(d):
                dv = jnp.full((16,), d, jnp.int32)
                for j in range(D // 16):
                    c = (dv + bases[j]) & (_L - 1)
                    v = plsc.load_gather(in_ref, [bases[j], c])
                    plsc.store_scatter(w_ref, [c, bases[j]], v * scale)

        in_copy(wid, 0).start()
        in_copy(wid + _NW, 1).start()

        def pair_step(g, carry):
            for slot in range(2):
                blk = wid + _NW * (2 * g + slot)
                in_copy(blk, slot).wait()

                @pl.when(g > 0)
                def _():
                    out_copy(0, _L, slot).wait()

                transpose(ins[slot], wbs[slot])
                out_copy(blk * _L, _L, slot).start()

                @pl.when(g + 1 < pairs)
                def _():
                    in_copy(blk + 2 * _NW, slot).start()

            return carry

        lax.fori_loop(0, pairs, pair_step, 0)
        out_copy(0, _L, 0).wait()
        out_copy(0, _L, 1).wait()

        if odd:
            blk = wid + _NW * (2 * pairs)
            in_copy(blk, 0).start()
            in_copy(blk, 0).wait()
            transpose(ins[0], wbs[0])
            out_copy(blk * _L, _L, 0).start()
            out_copy(0, _L, 0).wait()

        if extra:
            @pl.when(wid < extra)
            def _():
                blk = _NW * base + wid
                in_copy(blk, 0).start()
                in_copy(blk, 0).wait()
                transpose(ins[0], wbs[0])
                out_copy(blk * _L, _L, 0).start()
                out_copy(0, _L, 0).wait()

        if rem:
            @pl.when(wid == extra % _NW)
            def _():
                tc = pltpu.make_async_copy(tail_hbm, tbuf, gsems[1])
                tc.start()
                tc.wait()

                @plsc.parallel_loop(0, rem)
                def _(i):
                    for j in range(D // 16):
                        wbs[1][i, pl.ds(j * 16, 16)] = tbuf[i, pl.ds(j * 16, 16)]

                out_copy(V - rem, rem, 1).start()
                out_copy(0, rem, 1).wait()

    return body


@functools.lru_cache(maxsize=None)
def _build_gather(N, D, V):
    per_w = N // _NW
    n_chunks = per_w // _C
    n_super = n_chunks // _RING
    mesh = plsc.VectorSubcoreMesh(core_axis_name="c", subcore_axis_name="s")

    @functools.partial(
        pl.kernel,
        mesh=mesh,
        compiler_params=pltpu.CompilerParams(use_tc_tiling_on_sc=True),
        out_type=jax.ShapeDtypeStruct((N, D), jnp.float32),
        scratch_types=(
            [pltpu.VMEM((per_w,), jnp.int32)]
            + [pltpu.VMEM((_C, _L), jnp.float32) for _ in range(_RING)]
            + [pltpu.VMEM((_C, D), jnp.float32) for _ in range(_RING)]
            + [pltpu.SemaphoreType.DMA for _ in range(2 * _RING)]
        ),
    )
    def body(x_hbm, w_hbm, out_hbm, idx_v, *rest):
        rows = rest[:_RING]
        stage = rest[_RING:2 * _RING]
        gsems = rest[2 * _RING:3 * _RING]
        ssems = rest[3 * _RING:]
        wid = lax.axis_index("s") * _NC + lax.axis_index("c")
        base = wid * per_w

        pltpu.sync_copy(x_hbm.at[pl.ds(base, per_w)], idx_v)

        def g_copy(chunk, slot):
            return pltpu.make_async_copy(
                w_hbm.at[idx_v.at[pl.ds(chunk * _C, _C)]],
                rows[slot], gsems[slot])

        def s_copy(chunk, slot):
            return pltpu.make_async_copy(
                stage[slot],
                out_hbm.at[pl.ds(base + chunk * _C, _C)], ssems[slot])

        def compact(slot):
            rbuf, sbuf = rows[slot], stage[slot]

            @plsc.parallel_loop(0, _C, unroll=8)
            def _(i):
                for j in range(D // 16):
                    sbuf[i, pl.ds(j * 16, 16)] = rbuf[i, pl.ds(j * 16, 16)]

        for slot in range(_RING):
            g_copy(slot, slot).start()

        def super_step(g, carry):
            c0 = g * _RING
            for slot in range(_RING):
                g_copy(c0 + slot, slot).wait()

                @pl.when(g > 0)
                def _():
                    s_copy(c0 + slot, slot).wait()

                compact(slot)
                s_copy(c0 + slot, slot).start()

                @pl.when(g + 1 < n_super)
                def _():
                    g_copy(c0 + _RING + slot, slot).start()

            return carry

        lax.fori_loop(0, n_super, super_step, 0)

        for slot in range(_RING):
            s_copy(n_chunks - _RING + slot, slot).wait()

    return body


def kernel(x, table):
    B, S = x.shape
    V, D = table.shape
    N = B * S
    xf = x.reshape(N).astype(jnp.int32)
    rem = V % _L
    args = (table.T,)
    if rem:
        args += (table[V - rem:, :] * jnp.float32(float(D) ** -0.5),)
    w = _build_transpose(V, D)(*args)
    out = _build_gather(N, D, V)(xf, w)
    return out.reshape(B, S, D)
